# Initial kernel scaffold; baseline (speedup 1.0000x reference)
import jax
import jax.numpy as jnp
from jax.experimental import pallas as pl


def kernel(node_attrs, node_feats, edge_attrs, edge_feats, edge_index, W_src, W_tgt, W_up, W_skip, R0, R1, R2, R3, D0, D1, W_res, W1_s, W1_v, W2_s, W2_v, alpha, beta):
    raise NotImplementedError("write your pallas kernel here")


# speedup vs baseline: 1.3201x; 1.3201x over previous
import functools

import jax
import jax.numpy as jnp
from jax.experimental import pallas as pl

N_BLK = 1000
E_BLK = 2560


def _node_prep_body(nf_ref, na_ref, wup_ref, wa_ref, wb_ref, wskip_ref,
                    wupres_ref, usrc_ref, udst_ref, scs_ref, nfres_ref):
    nf = nf_ref[...]
    na = na_ref[...]
    nfu = jnp.dot(nf, wup_ref[...], preferred_element_type=jnp.float32)
    a = jnp.dot(na, wa_ref[...], preferred_element_type=jnp.float32)
    usrc_ref[...] = jnp.concatenate([nfu, a], axis=1)
    udst_ref[...] = jnp.dot(na, wb_ref[...], preferred_element_type=jnp.float32)
    scs_ref[...] = jnp.dot(nf, wskip_ref[...], preferred_element_type=jnp.float32)
    nfres_ref[...] = jnp.dot(nf, wupres_ref[...], preferred_element_type=jnp.float32)


def _node_prep(node_feats, node_attrs, W_up, WA, WB, W_skip, W_upres):
    n = node_feats.shape[0]
    d_attr = node_attrs.shape[1]
    grid = (n // N_BLK,)
    row = lambda i: (i, 0)
    fixed = lambda i: (0, 0)
    return pl.pallas_call(
        _node_prep_body,
        grid=grid,
        in_specs=[
            pl.BlockSpec((N_BLK, 128), row),
            pl.BlockSpec((N_BLK, d_attr), row),
            pl.BlockSpec((128, 128), fixed),
            pl.BlockSpec((d_attr, 128), fixed),
            pl.BlockSpec((d_attr, 128), fixed),
            pl.BlockSpec((128, 128), fixed),
            pl.BlockSpec((128, 256), fixed),
        ],
        out_specs=[
            pl.BlockSpec((N_BLK, 256), row),
            pl.BlockSpec((N_BLK, 128), row),
            pl.BlockSpec((N_BLK, 128), row),
            pl.BlockSpec((N_BLK, 256), row),
        ],
        out_shape=[
            jax.ShapeDtypeStruct((n, 256), jnp.float32),
            jax.ShapeDtypeStruct((n, 128), jnp.float32),
            jax.ShapeDtypeStruct((n, 128), jnp.float32),
            jax.ShapeDtypeStruct((n, 256), jnp.float32),
        ],
    )(node_feats, node_attrs, W_up, WA, WB, W_skip, W_upres)


def _edge_body(gsrc_ref, gdst_ref, ef_ref, ea_ref, r0e_ref, d0e_ref,
               r1_ref, r2_ref, r3_ref, d1_ref, mji_ref, dens_ref):
    gsrc = gsrc_ref[...]
    gdst = gdst_ref[...]
    ef = ef_ref[...]
    ea = ea_ref[...]
    x = gsrc[:, :128]
    h = ef @ r0e_ref[...] + gsrc[:, 128:192] + gdst[:, :64]
    h = jax.nn.silu(h)
    h = jax.nn.silu(jnp.dot(h, r1_ref[...], preferred_element_type=jnp.float32))
    h = jax.nn.silu(jnp.dot(h, r2_ref[...], preferred_element_type=jnp.float32))
    tpw = jnp.dot(h, r3_ref[...], preferred_element_type=jnp.float32)
    hd = ef @ d0e_ref[...] + gsrc[:, 192:256] + gdst[:, 64:128]
    hd = jax.nn.silu(hd)
    d = jnp.sum(hd * d1_ref[...], axis=1, keepdims=True)
    dens = jnp.tanh(d * d)
    dens_ref[...] = jnp.pad(dens, ((0, 0), (0, 7)))
    w0 = tpw[:, :128]
    w1 = tpw[:, 128:]
    sh0 = ea[:, 0:1]
    xw1 = x * w1
    m0 = x * sh0 * w0
    mx = xw1 * ea[:, 1:2]
    my = xw1 * ea[:, 2:3]
    mz = xw1 * ea[:, 3:4]
    mji_ref[...] = jnp.concatenate([m0, mx, my, mz], axis=1)


def _edge_stage(gsrc, gdst, edge_feats, edge_attrs, R0e, D0e, R1, R2, R3, D1):
    e = edge_feats.shape[0]
    grid = (e // E_BLK,)
    row = lambda i: (i, 0)
    fixed = lambda i: (0, 0)
    return pl.pallas_call(
        _edge_body,
        grid=grid,
        in_specs=[
            pl.BlockSpec((E_BLK, 256), row),
            pl.BlockSpec((E_BLK, 128), row),
            pl.BlockSpec((E_BLK, 8), row),
            pl.BlockSpec((E_BLK, 4), row),
            pl.BlockSpec((8, 64), fixed),
            pl.BlockSpec((8, 64), fixed),
            pl.BlockSpec((64, 64), fixed),
            pl.BlockSpec((64, 64), fixed),
            pl.BlockSpec((64, 256), fixed),
            pl.BlockSpec((1, 64), fixed),
        ],
        out_specs=[
            pl.BlockSpec((E_BLK, 512), row),
            pl.BlockSpec((E_BLK, 8), row),
        ],
        out_shape=[
            jax.ShapeDtypeStruct((e, 512), jnp.float32),
            jax.ShapeDtypeStruct((e, 8), jnp.float32),
        ],
    )(gsrc, gdst, edge_feats, edge_attrs, R0e, D0e, R1, R2, R3, D1)


def _final_body(msg_ref, dens_ref, nfres_ref, w1s_ref, w1v_ref, w2s_ref,
                w2v_ref, ab_ref, os_ref, ox_ref, oy_ref, oz_ref):
    msg = msg_ref[...]
    nfres = nfres_ref[...]
    alpha = ab_ref[0, 0]
    beta = ab_ref[0, 1]
    inv = 1.0 / (dens_ref[...][:, 0:1] * beta + alpha)
    msg_s = jnp.dot(msg[:, :128], w1s_ref[...], preferred_element_type=jnp.float32)
    scal = msg_s[:, :128] * inv + nfres[:, :128]
    gates = msg_s[:, 128:] * inv + nfres[:, 128:]
    os_ref[...] = jnp.dot(jax.nn.silu(scal), w2s_ref[...],
                          preferred_element_type=jnp.float32)
    g = jax.nn.sigmoid(gates)
    w1v = w1v_ref[...]
    w2v = w2v_ref[...]
    vx = jnp.dot(msg[:, 128:256], w1v, preferred_element_type=jnp.float32) * inv
    vy = jnp.dot(msg[:, 256:384], w1v, preferred_element_type=jnp.float32) * inv
    vz = jnp.dot(msg[:, 384:512], w1v, preferred_element_type=jnp.float32) * inv
    ox_ref[...] = jnp.dot(g * vx, w2v, preferred_element_type=jnp.float32)
    oy_ref[...] = jnp.dot(g * vy, w2v, preferred_element_type=jnp.float32)
    oz_ref[...] = jnp.dot(g * vz, w2v, preferred_element_type=jnp.float32)


def _final_stage(message, dens, nfres, W1_s, W1_v, W2_s, W2_v, ab):
    n = message.shape[0]
    grid = (n // N_BLK,)
    row = lambda i: (i, 0)
    fixed = lambda i: (0, 0)
    return pl.pallas_call(
        _final_body,
        grid=grid,
        in_specs=[
            pl.BlockSpec((N_BLK, 512), row),
            pl.BlockSpec((N_BLK, 8), row),
            pl.BlockSpec((N_BLK, 256), row),
            pl.BlockSpec((128, 256), fixed),
            pl.BlockSpec((128, 128), fixed),
            pl.BlockSpec((128, 128), fixed),
            pl.BlockSpec((128, 128), fixed),
            pl.BlockSpec((1, 2), fixed),
        ],
        out_specs=[pl.BlockSpec((N_BLK, 128), row)] * 4,
        out_shape=[jax.ShapeDtypeStruct((n, 128), jnp.float32)] * 4,
    )(message, dens, nfres, W1_s, W1_v, W2_s, W2_v, ab)


def kernel(node_attrs, node_feats, edge_attrs, edge_feats, edge_index,
           W_src, W_tgt, W_up, W_skip, R0, R1, R2, R3, D0, D1,
           W_res, W1_s, W1_v, W2_s, W2_v, alpha, beta):
    n = node_feats.shape[0]
    e = edge_attrs.shape[0]
    src = edge_index[0]
    dst = edge_index[1]
    WA = jnp.concatenate([W_src @ R0[8:136], W_src @ D0[8:136]], axis=1)
    WB = jnp.concatenate([W_tgt @ R0[136:264], W_tgt @ D0[136:264]], axis=1)
    W_upres = W_up @ W_res
    R0e = R0[:8]
    D0e = D0[:8]

    usrc, udst, sc_s, nfres = _node_prep(node_feats, node_attrs, W_up, WA, WB,
                                         W_skip, W_upres)

    gsrc = jnp.take(usrc, src, axis=0)
    gdst = jnp.take(udst, dst, axis=0)

    mji, dens_e = _edge_stage(gsrc, gdst, edge_feats, edge_attrs,
                              R0e, D0e, R1, R2, R3, D1.reshape(1, 64))

    message = jax.ops.segment_sum(mji, dst, num_segments=n)
    dens = jax.ops.segment_sum(dens_e, dst, num_segments=n)

    ab = jnp.stack([alpha, beta]).reshape(1, 2)
    o_s, o_x, o_y, o_z = _final_stage(message, dens, nfres, W1_s, W1_v,
                                      W2_s, W2_v, ab)

    reshaped = jnp.stack([o_s, o_x, o_y, o_z], axis=-1)
    sc = jnp.concatenate([sc_s, jnp.zeros((n, 384), jnp.float32)], axis=1)
    return (reshaped, sc)
